# Initial kernel scaffold; baseline (speedup 1.0000x reference)
import jax
import jax.numpy as jnp
from jax.experimental import pallas as pl


def kernel(hidden_states, router_logits, up_weight, down_weight):
    raise NotImplementedError("write your pallas kernel here")


# speedup vs baseline: 1.1493x; 1.1493x over previous
import functools

import jax
import jax.numpy as jnp
from jax.experimental import pallas as pl
from jax.experimental.pallas import tpu as pltpu

_TOPK = 2


def _moe_body(rl_ref, hs_ref, gw_ref, uw_ref, dw_ref, out_ref):
    e = pl.program_id(0)
    i = pl.program_id(1)

    @pl.when((e == 0) & (i == 0))
    def _init():
        out_ref[...] = jnp.zeros_like(out_ref)

    g = jnp.dot(hs_ref[...], gw_ref[0], preferred_element_type=jnp.float32)
    u = jnp.dot(hs_ref[...], uw_ref[0], preferred_element_type=jnp.float32)
    act = (g * jax.nn.sigmoid(g)) * u
    y = jnp.dot(act.astype(jnp.bfloat16), dw_ref[0],
                preferred_element_type=jnp.float32)

    logits = rl_ref[...]
    T, E = logits.shape
    col = jax.lax.broadcasted_iota(jnp.int32, (T, E), 1)
    m1 = jnp.max(logits, axis=1, keepdims=True)
    a1 = jnp.min(jnp.where(logits == m1, col, E), axis=1, keepdims=True)
    masked = jnp.where(col == a1, -jnp.inf, logits)
    m2 = jnp.max(masked, axis=1, keepdims=True)
    a2 = jnp.min(jnp.where(masked == m2, col, E), axis=1, keepdims=True)
    z = jnp.exp(m2 - m1)
    w1 = 1.0 / (1.0 + z)
    w2 = z * w1
    w = w1 * (a1 == e).astype(jnp.float32) + w2 * (a2 == e).astype(jnp.float32)

    out_ref[...] += y * w


@jax.jit
def kernel(hidden_states, router_logits, up_weight, down_weight):
    T, H = hidden_states.shape
    E = up_weight.shape[0]
    I = down_weight.shape[1]
    TI = 512
    NI = I // TI

    hs = hidden_states.astype(jnp.bfloat16)
    gate_w = up_weight[:, :, :I].astype(jnp.bfloat16)
    up_w = up_weight[:, :, I:].astype(jnp.bfloat16)
    dw = down_weight.astype(jnp.bfloat16)

    return pl.pallas_call(
        _moe_body,
        grid=(E, NI),
        in_specs=[
            pl.BlockSpec((T, E), lambda e, i: (0, 0)),
            pl.BlockSpec((T, H), lambda e, i: (0, 0)),
            pl.BlockSpec((1, H, TI), lambda e, i: (e, 0, i)),
            pl.BlockSpec((1, H, TI), lambda e, i: (e, 0, i)),
            pl.BlockSpec((1, TI, H), lambda e, i: (e, i, 0)),
        ],
        out_specs=pl.BlockSpec((T, H), lambda e, i: (0, 0)),
        out_shape=jax.ShapeDtypeStruct((T, H), jnp.float32),
        compiler_params=pltpu.CompilerParams(
            dimension_semantics=("arbitrary", "arbitrary"),
        ),
    )(router_logits, hs, gate_w, up_w, dw)


# speedup vs baseline: 1.2697x; 1.1048x over previous
import functools

import jax
import jax.numpy as jnp
from jax.experimental import pallas as pl
from jax.experimental.pallas import tpu as pltpu

_TOPK = 2


def _moe_body(rl_ref, hs_ref, gw_ref, uw_ref, dw_ref, out_ref, w_scr):
    e = pl.program_id(0)
    i = pl.program_id(1)

    @pl.when((e == 0) & (i == 0))
    def _init():
        out_ref[...] = jnp.zeros_like(out_ref)
        logits = rl_ref[...]
        T, E = logits.shape
        col = jax.lax.broadcasted_iota(jnp.int32, (T, E), 1)
        m1 = jnp.max(logits, axis=1, keepdims=True)
        a1 = jnp.min(jnp.where(logits == m1, col, E), axis=1, keepdims=True)
        masked = jnp.where(col == a1, -jnp.inf, logits)
        m2 = jnp.max(masked, axis=1, keepdims=True)
        a2 = jnp.min(jnp.where(masked == m2, col, E), axis=1, keepdims=True)
        z = jnp.exp(m2 - m1)
        w1 = 1.0 / (1.0 + z)
        w2 = z * w1
        w_scr[...] = (w1 * (col == a1).astype(jnp.float32)
                      + w2 * (col == a2).astype(jnp.float32))

    g = jnp.dot(hs_ref[...], gw_ref[0], preferred_element_type=jnp.float32)
    u = jnp.dot(hs_ref[...], uw_ref[0], preferred_element_type=jnp.float32)
    act = (g * jax.nn.sigmoid(g)) * u
    y = jnp.dot(act.astype(jnp.bfloat16), dw_ref[0],
                preferred_element_type=jnp.float32)

    W = w_scr[...]
    col = jax.lax.broadcasted_iota(jnp.int32, W.shape, 1)
    w = jnp.sum(jnp.where(col == e, W, 0.0), axis=1, keepdims=True)
    out_ref[...] += y * w


@jax.jit
def kernel(hidden_states, router_logits, up_weight, down_weight):
    T, H = hidden_states.shape
    E = up_weight.shape[0]
    I = down_weight.shape[1]
    TI = 512
    NI = I // TI

    hs = hidden_states.astype(jnp.bfloat16)
    gate_w = up_weight[:, :, :I].astype(jnp.bfloat16)
    up_w = up_weight[:, :, I:].astype(jnp.bfloat16)
    dw = down_weight.astype(jnp.bfloat16)

    return pl.pallas_call(
        _moe_body,
        grid=(E, NI),
        in_specs=[
            pl.BlockSpec((T, E), lambda e, i: (0, 0)),
            pl.BlockSpec((T, H), lambda e, i: (0, 0)),
            pl.BlockSpec((1, H, TI), lambda e, i: (e, 0, i)),
            pl.BlockSpec((1, H, TI), lambda e, i: (e, 0, i)),
            pl.BlockSpec((1, TI, H), lambda e, i: (e, i, 0)),
        ],
        out_specs=pl.BlockSpec((T, H), lambda e, i: (0, 0)),
        out_shape=jax.ShapeDtypeStruct((T, H), jnp.float32),
        scratch_shapes=[pltpu.VMEM((T, E), jnp.float32)],
        compiler_params=pltpu.CompilerParams(
            dimension_semantics=("arbitrary", "arbitrary"),
        ),
    )(router_logits, hs, gate_w, up_w, dw)
